# Initial kernel scaffold; baseline (speedup 1.0000x reference)
import jax
import jax.numpy as jnp
from jax.experimental import pallas as pl
from jax.experimental.pallas import tpu as pltpu
from jax.experimental.pallas import tpu_sc as plsc

_B, _T, _D, _K = 16, 576, 256, 8192
_TAU = 1.0
_N = _B * _T
_TM = 256
_GRID = _N // _TM
_GW = 128


def _vq_body(l2_ref, lat_ref, cbt_ref, y_ref, rep_ref, hy_ref, cnt_ref):
    i = pl.program_id(0)
    lat = lat_ref[...]
    cbt = cbt_ref[...]
    lc = jax.lax.dot_general(
        lat, cbt, (((1,), (0,)), ((), ())),
        preferred_element_type=jnp.float32)
    c2 = jnp.sum(cbt * cbt, axis=0, keepdims=True)
    scores = -0.5 * (l2_ref[...] - 2.0 * lc + c2) / _TAU
    y = jnp.argmax(scores, axis=-1).astype(jnp.int32)
    y2 = y.reshape(_TM, 1)
    y_ref[...] = y2
    ids = jax.lax.broadcasted_iota(jnp.int32, (_TM, _K), 1)
    rep = (ids == y2).astype(jnp.float32)
    rep_ref[...] = rep
    cnt = jnp.sum(rep, axis=0, keepdims=True)

    @pl.when(i == 0)
    def _():
        cnt_ref[...] = cnt

    @pl.when(i > 0)
    def _():
        cnt_ref[...] = cnt_ref[...] + cnt

    @pl.when(i == _GRID - 1)
    def _():
        py = cnt_ref[...] / jnp.float32(_N)
        hy_ref[0, 0] = -jnp.sum(py * jnp.log2(py + 1e-10))


def _vq_tc(l2, lat, cbt):
    return pl.pallas_call(
        _vq_body,
        grid=(_GRID,),
        in_specs=[
            pl.BlockSpec((_TM, 1), lambda i: (i, 0)),
            pl.BlockSpec((_TM, _D), lambda i: (i, 0)),
            pl.BlockSpec((_D, _K), lambda i: (0, 0)),
        ],
        out_specs=[
            pl.BlockSpec((_TM, 1), lambda i: (i, 0)),
            pl.BlockSpec((_TM, _K), lambda i: (i, 0)),
            pl.BlockSpec((1, 1), lambda i: (0, 0)),
        ],
        out_shape=[
            jax.ShapeDtypeStruct((_N, 1), jnp.int32),
            jax.ShapeDtypeStruct((_N, _K), jnp.float32),
            jax.ShapeDtypeStruct((1, 1), jnp.float32),
        ],
        scratch_shapes=[pltpu.VMEM((1, _K), jnp.float32)],
    )(l2, lat, cbt)


def _sc_gather(codebook, idx_row):
    @pl.kernel(
        out_type=jax.ShapeDtypeStruct((_N, _D), jnp.float32),
        mesh=plsc.VectorSubcoreMesh(
            core_axis_name="core", subcore_axis_name="subcore"),
    )
    def k(cb_hbm, i_hbm, o_hbm):
        def body(i_vmem, o_vmem):
            pltpu.sync_copy(cb_hbm.at[i_vmem.at[0]], o_vmem)

        pltpu.emit_pipeline(
            body,
            grid=(_N // _GW,),
            in_specs=[pl.BlockSpec((1, _GW), index_map=lambda i: (0, i))],
            out_specs=[pl.BlockSpec((_GW, _D), index_map=lambda i: (i, 0))],
            core_axis_name=("core", "subcore"),
            dimension_semantics=(pltpu.PARALLEL,),
        )(i_hbm, o_hbm)

    return k(codebook, idx_row)


def kernel(query, codebook):
    latent = query.reshape(_B, -1, _D)
    l2 = jnp.sum(latent * latent, axis=-1, keepdims=True)
    lat2d = latent.reshape(_N, _D)
    y2d, rep2d, hy = _vq_tc(l2.reshape(_N, 1), lat2d, codebook.T)
    y = y2d.reshape(_B, _T)
    representation = rep2d.reshape(_B, _T, _K)
    quant = _sc_gather(codebook, y2d.reshape(1, _N))
    embedding = quant.reshape(_B, _T, _D)
    Hy = hy[0, 0]
    Hyx = jnp.zeros((), jnp.float32)
    return (latent, embedding, y, representation, Hyx, Hy)


# speedup vs baseline: 1.1336x; 1.1336x over previous
import jax
import jax.numpy as jnp
from jax.experimental import pallas as pl
from jax.experimental.pallas import tpu as pltpu
from jax.experimental.pallas import tpu_sc as plsc

_B, _T, _D, _K = 16, 576, 256, 8192
_TAU = 1.0
_N = _B * _T
_TM = 256
_GRID = _N // _TM
_GW = 128


def _vq_body(l2_ref, lat_ref, cbt_ref, y_ref, rep_ref, hy_ref, cnt_ref):
    i = pl.program_id(0)
    lat = lat_ref[...]
    cbt = cbt_ref[...]
    lc = jax.lax.dot_general(
        lat, cbt, (((1,), (0,)), ((), ())),
        precision=jax.lax.Precision.DEFAULT,
        preferred_element_type=jnp.float32)
    c2 = jnp.sum(cbt * cbt, axis=0, keepdims=True)
    scores = -0.5 * (l2_ref[...] - 2.0 * lc + c2) / _TAU
    ids = jax.lax.broadcasted_iota(jnp.int32, (_TM, _K), 1)
    m = jnp.max(scores, axis=-1, keepdims=True)
    y2 = jnp.min(jnp.where(scores == m, ids, _K), axis=-1,
                 keepdims=True).astype(jnp.int32)
    y_ref[...] = y2
    rep = (ids == y2).astype(jnp.float32)
    rep_ref[...] = rep
    cnt = jnp.sum(rep, axis=0, keepdims=True)

    @pl.when(i == 0)
    def _():
        cnt_ref[...] = cnt

    @pl.when(i > 0)
    def _():
        cnt_ref[...] = cnt_ref[...] + cnt

    @pl.when(i == _GRID - 1)
    def _():
        py = cnt_ref[...] / jnp.float32(_N)
        hy_ref[...] = -jnp.sum(py * jnp.log2(py + 1e-10), axis=1, keepdims=True)


def _vq_tc(l2, lat, cbt):
    return pl.pallas_call(
        _vq_body,
        grid=(_GRID,),
        in_specs=[
            pl.BlockSpec((_TM, 1), lambda i: (i, 0)),
            pl.BlockSpec((_TM, _D), lambda i: (i, 0)),
            pl.BlockSpec((_D, _K), lambda i: (0, 0)),
        ],
        out_specs=[
            pl.BlockSpec((_TM, 1), lambda i: (i, 0)),
            pl.BlockSpec((_TM, _K), lambda i: (i, 0)),
            pl.BlockSpec((1, 1), lambda i: (0, 0)),
        ],
        out_shape=[
            jax.ShapeDtypeStruct((_N, 1), jnp.int32),
            jax.ShapeDtypeStruct((_N, _K), jnp.float32),
            jax.ShapeDtypeStruct((1, 1), jnp.float32),
        ],
        scratch_shapes=[pltpu.VMEM((1, _K), jnp.float32)],
    )(l2, lat, cbt)


def _sc_gather(codebook, idx_row):
    @pl.kernel(
        out_type=jax.ShapeDtypeStruct((_N, _D), jnp.float32),
        mesh=plsc.VectorSubcoreMesh(
            core_axis_name="core", subcore_axis_name="subcore"),
    )
    def k(cb_hbm, i_hbm, o_hbm):
        def body(i_vmem, o_vmem):
            pltpu.sync_copy(cb_hbm.at[i_vmem.at[0]], o_vmem)

        pltpu.emit_pipeline(
            body,
            grid=(_N // _GW,),
            in_specs=[pl.BlockSpec((1, _GW), index_map=lambda i: (0, i))],
            out_specs=[pl.BlockSpec((_GW, _D), index_map=lambda i: (i, 0))],
            core_axis_name=("core", "subcore"),
            dimension_semantics=(pltpu.PARALLEL,),
        )(i_hbm, o_hbm)

    return k(codebook, idx_row)


def kernel(query, codebook):
    latent = query.reshape(_B, -1, _D)
    l2 = jnp.sum(latent * latent, axis=-1, keepdims=True)
    lat2d = latent.reshape(_N, _D)
    y2d, rep2d, hy = _vq_tc(l2.reshape(_N, 1), lat2d, codebook.T)
    y = y2d.reshape(_B, _T)
    representation = rep2d.reshape(_B, _T, _K)
    quant = _sc_gather(codebook, y2d.reshape(1, _N))
    embedding = quant.reshape(_B, _T, _D)
    Hy = hy[0, 0]
    Hyx = jnp.zeros((), jnp.float32)
    return (latent, embedding, y, representation, Hyx, Hy)


# speedup vs baseline: 1.4302x; 1.2616x over previous
import jax
import jax.numpy as jnp
from jax.experimental import pallas as pl
from jax.experimental.pallas import tpu as pltpu
from jax.experimental.pallas import tpu_sc as plsc

_B, _T, _D, _K = 16, 576, 256, 8192
_TAU = 1.0
_N = _B * _T
_TM = 256
_GRID = _N // _TM
_GW = 128


def _vq_body(l2_ref, lat_ref, cbt_ref, c2_ref, y_ref, rep_ref, hy_ref,
             ids_ref, cnt_ref):
    i = pl.program_id(0)

    @pl.when(i == 0)
    def _():
        ids_ref[...] = jax.lax.broadcasted_iota(
            jnp.int32, (1, _K), 1).astype(jnp.float32)

    lat = lat_ref[...]
    lc2 = jax.lax.dot_general(
        lat + lat, cbt_ref[...], (((1,), (0,)), ((), ())),
        precision=jax.lax.Precision.DEFAULT,
        preferred_element_type=jnp.float32)
    t = (l2_ref[...] - lc2) + c2_ref[...]
    m = jnp.min(t, axis=-1, keepdims=True)
    idsf = ids_ref[...]
    y2f = jnp.min(jnp.where(t == m, idsf, jnp.float32(_K)),
                  axis=-1, keepdims=True)
    y_ref[...] = y2f.astype(jnp.int32)
    rep = (idsf == y2f).astype(jnp.float32)
    rep_ref[...] = rep
    cnt = jnp.sum(rep, axis=0, keepdims=True)

    @pl.when(i == 0)
    def _():
        cnt_ref[...] = cnt

    @pl.when(i > 0)
    def _():
        cnt_ref[...] = cnt_ref[...] + cnt

    @pl.when(i == _GRID - 1)
    def _():
        py = cnt_ref[...] / jnp.float32(_N)
        hy_ref[...] = -jnp.sum(py * jnp.log2(py + 1e-10), axis=1, keepdims=True)


def _vq_tc(l2, lat, cbt, c2):
    return pl.pallas_call(
        _vq_body,
        grid=(_GRID,),
        in_specs=[
            pl.BlockSpec((_TM, 1), lambda i: (i, 0)),
            pl.BlockSpec((_TM, _D), lambda i: (i, 0)),
            pl.BlockSpec((_D, _K), lambda i: (0, 0)),
            pl.BlockSpec((1, _K), lambda i: (0, 0)),
        ],
        out_specs=[
            pl.BlockSpec((_TM, 1), lambda i: (i, 0)),
            pl.BlockSpec((_TM, _K), lambda i: (i, 0)),
            pl.BlockSpec((1, 1), lambda i: (0, 0)),
        ],
        out_shape=[
            jax.ShapeDtypeStruct((_N, 1), jnp.int32),
            jax.ShapeDtypeStruct((_N, _K), jnp.float32),
            jax.ShapeDtypeStruct((1, 1), jnp.float32),
        ],
        scratch_shapes=[pltpu.VMEM((1, _K), jnp.float32),
                        pltpu.VMEM((1, _K), jnp.float32)],
    )(l2, lat, cbt, c2)


def _sc_gather(codebook, idx_row):
    @pl.kernel(
        out_type=jax.ShapeDtypeStruct((_N, _D), jnp.float32),
        mesh=plsc.VectorSubcoreMesh(
            core_axis_name="core", subcore_axis_name="subcore"),
    )
    def k(cb_hbm, i_hbm, o_hbm):
        def body(i_vmem, o_vmem):
            pltpu.sync_copy(cb_hbm.at[i_vmem.at[0]], o_vmem)

        pltpu.emit_pipeline(
            body,
            grid=(_N // _GW,),
            in_specs=[pl.BlockSpec((1, _GW), index_map=lambda i: (0, i))],
            out_specs=[pl.BlockSpec((_GW, _D), index_map=lambda i: (i, 0))],
            core_axis_name=("core", "subcore"),
            dimension_semantics=(pltpu.PARALLEL,),
        )(i_hbm, o_hbm)

    return k(codebook, idx_row)


def kernel(query, codebook):
    latent = query.reshape(_B, -1, _D)
    l2 = jnp.sum(latent * latent, axis=-1, keepdims=True)
    c2 = jnp.sum(codebook * codebook, axis=-1)
    lat2d = latent.reshape(_N, _D)
    y2d, rep2d, hy = _vq_tc(l2.reshape(_N, 1), lat2d, codebook.T,
                            c2.reshape(1, _K))
    y = y2d.reshape(_B, _T)
    representation = rep2d.reshape(_B, _T, _K)
    quant = _sc_gather(codebook, y2d.reshape(1, _N))
    embedding = quant.reshape(_B, _T, _D)
    Hy = hy[0, 0]
    Hyx = jnp.zeros((), jnp.float32)
    return (latent, embedding, y, representation, Hyx, Hy)
